# baseline (speedup 1.0000x reference)
import functools

import jax
import jax.numpy as jnp
from jax import lax
from jax.experimental import pallas as pl
from jax.experimental.pallas import tpu as pltpu
from jax.experimental.pallas import tpu_sc as plsc

EOS = 1e-10
NNODES = 10000
NEDGES = 320000
IN_DIM = 128
HID = 128
ALPHA = 1.0
TEMP = 1.0
BIAS = 0.0001

LANE = 128
ROWS = NEDGES // LANE
NTILES = 32
TROWS = 80
RPAD = NTILES * TROWS

_f32 = jnp.float32
_i32 = jnp.int32


def _tc_prep_body(f_ref, w1_ref, b1_ref, we_ref, be_ref, eps_ref, q_ref, g_ref):
    h = jnp.dot(f_ref[...], w1_ref[...], preferred_element_type=_f32)
    h = jnp.maximum(h + b1_ref[...], 0.0)
    w2 = 0.5 * (we_ref[:HID, :] + we_ref[HID:, :])
    q_ref[...] = jnp.dot(h, w2, preferred_element_type=_f32)
    e = eps_ref[...]
    eb = (BIAS - (1.0 - BIAS)) * e + (1.0 - BIAS)
    g_ref[...] = jnp.log(eb) - jnp.log(1.0 - eb) + be_ref[0, 0]


_tc_prep = pl.pallas_call(
    _tc_prep_body,
    out_shape=(
        jax.ShapeDtypeStruct((NNODES, 1), _f32),
        jax.ShapeDtypeStruct((RPAD, LANE), _f32),
    ),
)


def _tc_norm_body(dlp0_ref, dlp1_ref, dhp0_ref, dhp1_ref,
                  ilp_ref, ihp_ref, tlp_ref):
    dl = dlp0_ref[...] + dlp1_ref[...] + (1.0 + EOS)
    dh = dhp0_ref[...] + dhp1_ref[...] + (1.0 + EOS)
    ilp_ref[...] = lax.rsqrt(dl)
    ihp_ref[...] = lax.rsqrt(dh)
    tlp_ref[...] = (1.0 + EOS) / dl


_tc_norm = pl.pallas_call(
    _tc_norm_body,
    out_shape=(
        jax.ShapeDtypeStruct((1, NNODES), _f32),
        jax.ShapeDtypeStruct((1, NNODES), _f32),
        jax.ShapeDtypeStruct((1, NNODES), _f32),
    ),
)


_MESH = plsc.VectorSubcoreMesh(core_axis_name="c", subcore_axis_name="s",
                               num_cores=2, num_subcores=16)
_CP = pltpu.CompilerParams(needs_layout_passes=False)


@functools.partial(
    pl.kernel,
    out_type=(
        jax.ShapeDtypeStruct((RPAD, LANE), _f32),
        jax.ShapeDtypeStruct((RPAD, LANE), _f32),
        jax.ShapeDtypeStruct((NNODES,), _f32),
        jax.ShapeDtypeStruct((NNODES,), _f32),
        jax.ShapeDtypeStruct((NNODES,), _f32),
        jax.ShapeDtypeStruct((NNODES,), _f32),
    ),
    mesh=_MESH,
    compiler_params=_CP,
    scratch_types=[
        pltpu.VMEM((NNODES,), _f32),
        pltpu.VMEM((NNODES,), _f32),
        pltpu.VMEM((TROWS, LANE), _i32),
        pltpu.VMEM((TROWS, LANE), _i32),
        pltpu.VMEM((TROWS, LANE), _f32),
        pltpu.VMEM((TROWS, LANE), _f32),
        pltpu.VMEM((TROWS, LANE), _f32),
        pltpu.VMEM((TROWS * LANE,), _f32),
        pltpu.VMEM((TROWS * LANE,), _f32),
        pltpu.VMEM((TROWS * LANE,), _i32),
        pltpu.VMEM_SHARED((NNODES,), _f32),
        pltpu.VMEM_SHARED((NNODES,), _f32),
        pltpu.SemaphoreType.DMA,
    ],
)
def _sc_weights(q_hbm, src_hbm, dst_hbm, g_hbm,
                wlp_hbm, whp_hbm, dlp0_hbm, dlp1_hbm, dhp0_hbm, dhp1_hbm,
                q_v, z_v, src_v, dst_v, g_v, wlp_v, whp_v, wlpe_v, whpe_v,
                dstf_v, sh_lp, sh_hp, sem):
    c = lax.axis_index("c")
    s = lax.axis_index("s")
    wid = s * 2 + c
    start = wid * TROWS
    nrows = jnp.minimum(TROWS, ROWS - start)

    cp0 = pltpu.async_copy(q_hbm, q_v, sem)
    cp1 = pltpu.async_copy(src_hbm.at[pl.ds(start, TROWS)], src_v, sem)
    cp2 = pltpu.async_copy(dst_hbm.at[pl.ds(start, TROWS)], dst_v, sem)
    cp3 = pltpu.async_copy(g_hbm.at[pl.ds(start, TROWS)], g_v, sem)

    @pl.when(s == 0)
    def _init_shared():
        def zbody(i, carry):
            z_v[pl.ds(i * 16, 16)] = jnp.zeros((16,), _f32)
            return carry
        lax.fori_loop(0, NNODES // 16, zbody, 0)
        pltpu.sync_copy(z_v, sh_lp)
        pltpu.sync_copy(z_v, sh_hp)

    cp0.wait()
    cp1.wait()
    cp2.wait()
    cp3.wait()
    plsc.subcore_barrier()

    def row_body(j, carry):
        m = (j < nrows).astype(_f32)
        for k in range(LANE // 16):
            sl = pl.ds(k * 16, 16)
            si = src_v[j, sl]
            di = dst_v[j, sl]
            qs = plsc.load_gather(q_v, [si])
            qd = plsc.load_gather(q_v, [di])
            x = (g_v[j, sl] + qs + qd) / TEMP
            w = 1.0 / (1.0 + jnp.exp(-x))
            wlp_v[j, sl] = w
            whp_v[j, sl] = 1.0 - w
            fl = pl.ds(j * LANE + k * 16, 16)
            wlpe_v[fl] = (w + EOS) * m
            whpe_v[fl] = ((1.0 - w) + EOS) * m
            dstf_v[fl] = di
        return carry

    lax.fori_loop(0, TROWS, row_body, 0)

    pltpu.sync_copy(wlpe_v, sh_lp.at[dstf_v], add=True)
    pltpu.sync_copy(whpe_v, sh_hp.at[dstf_v], add=True)

    pltpu.sync_copy(wlp_v, wlp_hbm.at[pl.ds(start, TROWS)])
    pltpu.sync_copy(whp_v, whp_hbm.at[pl.ds(start, TROWS)])

    plsc.subcore_barrier()

    @pl.when((s == 0) & (c == 0))
    def _writeback_c0():
        pltpu.sync_copy(sh_lp, dlp0_hbm)
        pltpu.sync_copy(sh_hp, dhp0_hbm)

    @pl.when((s == 0) & (c == 1))
    def _writeback_c1():
        pltpu.sync_copy(sh_lp, dlp1_hbm)
        pltpu.sync_copy(sh_hp, dhp1_hbm)


@functools.partial(
    pl.kernel,
    out_type=(
        jax.ShapeDtypeStruct((RPAD, LANE), _f32),
        jax.ShapeDtypeStruct((RPAD, LANE), _f32),
    ),
    mesh=_MESH,
    compiler_params=_CP,
    scratch_types=[
        pltpu.VMEM((NNODES,), _f32),
        pltpu.VMEM((NNODES,), _f32),
        pltpu.VMEM((TROWS, LANE), _i32),
        pltpu.VMEM((TROWS, LANE), _i32),
        pltpu.VMEM((TROWS, LANE), _f32),
        pltpu.VMEM((TROWS, LANE), _f32),
        pltpu.VMEM((TROWS, LANE), _f32),
        pltpu.VMEM((TROWS, LANE), _f32),
        pltpu.VMEM((TROWS * LANE,), _i32),
        pltpu.VMEM((TROWS * LANE,), _f32),
        pltpu.SemaphoreType.DMA,
    ],
)
def _sc_norm_scatter(ilp_hbm, ihp_hbm, src_hbm, dst_hbm, wlp_hbm, whp_hbm,
                     adj_hbm, olp_hbm, ohp_hbm,
                     ilp_v, ihp_v, src_v, dst_v, wlp_v, whp_v,
                     olp_v, ohp_v, fidx_v, ones_v, sem):
    c = lax.axis_index("c")
    s = lax.axis_index("s")
    wid = s * 2 + c
    start = wid * TROWS
    nrows = jnp.minimum(TROWS, ROWS - start)

    cp0 = pltpu.async_copy(ilp_hbm, ilp_v, sem)
    cp1 = pltpu.async_copy(ihp_hbm, ihp_v, sem)
    cp2 = pltpu.async_copy(src_hbm.at[pl.ds(start, TROWS)], src_v, sem)
    cp3 = pltpu.async_copy(dst_hbm.at[pl.ds(start, TROWS)], dst_v, sem)
    cp4 = pltpu.async_copy(wlp_hbm.at[pl.ds(start, TROWS)], wlp_v, sem)
    cp5 = pltpu.async_copy(whp_hbm.at[pl.ds(start, TROWS)], whp_v, sem)
    cp0.wait()
    cp1.wait()
    cp2.wait()
    cp3.wait()
    cp4.wait()
    cp5.wait()

    def row_body(j, carry):
        jj = jnp.minimum(j, nrows - 1)
        for k in range(LANE // 16):
            sl = pl.ds(k * 16, 16)
            si = src_v[jj, sl]
            di = dst_v[jj, sl]
            ils = plsc.load_gather(ilp_v, [si])
            ild = plsc.load_gather(ilp_v, [di])
            ihs = plsc.load_gather(ihp_v, [si])
            ihd = plsc.load_gather(ihp_v, [di])
            olp_v[j, sl] = (wlp_v[jj, sl] + EOS) * ils * ild
            ohp_v[j, sl] = (-ALPHA) * ((whp_v[jj, sl] + EOS) * ihs * ihd)
            fl = pl.ds(j * LANE + k * 16, 16)
            fidx_v[fl] = si * NNODES + di
            ones_v[fl] = jnp.full((16,), 1.0, _f32)
        return carry

    lax.fori_loop(0, TROWS, row_body, 0)

    pltpu.sync_copy(ones_v, adj_hbm.at[fidx_v])

    pltpu.sync_copy(olp_v, olp_hbm.at[pl.ds(start, TROWS)])
    pltpu.sync_copy(ohp_v, ohp_hbm.at[pl.ds(start, TROWS)])


def kernel(features, edges, eps, W1, b1, W_edge, b_edge):
    src = edges[0].astype(_i32)
    dst = edges[1].astype(_i32)
    pad = ((0, RPAD - ROWS), (0, 0))
    src2 = jnp.pad(src.reshape(ROWS, LANE), pad)
    dst2 = jnp.pad(dst.reshape(ROWS, LANE), pad)
    eps2 = jnp.pad(eps.reshape(ROWS, LANE), pad)

    q2, g2 = _tc_prep(features, W1, b1.reshape(1, HID), W_edge,
                      b_edge.reshape(1, 1), eps2)
    q = q2.reshape(NNODES)

    wlp2, whp2, dlp0, dlp1, dhp0, dhp1 = _sc_weights(q, src2, dst2, g2)
    ilp, ihp, tlp = _tc_norm(dlp0.reshape(1, NNODES), dlp1.reshape(1, NNODES),
                             dhp0.reshape(1, NNODES), dhp1.reshape(1, NNODES))

    adj_ref = jax.new_ref(jnp.zeros((NNODES * NNODES,), _f32))
    olp2, ohp2 = _sc_norm_scatter(ilp.reshape(NNODES), ihp.reshape(NNODES),
                                  src2, dst2, wlp2, whp2, adj_ref)
    adj = jax.freeze(adj_ref).reshape(NNODES, NNODES)

    weights_lp = wlp2[:ROWS].reshape(NEDGES)
    weights_hp = whp2[:ROWS].reshape(NEDGES)
    w_lp_norm = jnp.concatenate([olp2[:ROWS].reshape(NEDGES), tlp.reshape(NNODES)])
    w_hp_norm = jnp.concatenate([ohp2[:ROWS].reshape(NEDGES),
                                 jnp.ones((NNODES,), _f32)])
    return (w_lp_norm, w_hp_norm, weights_lp, weights_hp, adj)


# speedup vs baseline: 1.7970x; 1.7970x over previous
import functools

import jax
import jax.numpy as jnp
from jax import lax
from jax.experimental import pallas as pl
from jax.experimental.pallas import tpu as pltpu
from jax.experimental.pallas import tpu_sc as plsc

EOS = 1e-10
NNODES = 10000
NEDGES = 320000
IN_DIM = 128
HID = 128
ALPHA = 1.0
TEMP = 1.0
BIAS = 0.0001

LANE = 128
ROWS = NEDGES // LANE
NTILES = 32
TROWS = 80
RPAD = NTILES * TROWS
EPT = TROWS * LANE

RT = 312
NU_MAX = 41
CAP1 = 64
CAP2 = 32
OVC = EPT // 16

_f32 = jnp.float32
_i32 = jnp.int32


def _tc_prep_body(f_ref, w1_ref, b1_ref, we_ref, be_ref, eps_ref, q_ref, g_ref):
    h = jnp.dot(f_ref[...], w1_ref[...], preferred_element_type=_f32)
    h = jnp.maximum(h + b1_ref[...], 0.0)
    w2 = 0.5 * (we_ref[:HID, :] + we_ref[HID:, :])
    q_ref[...] = jnp.dot(h, w2, preferred_element_type=_f32)
    e = eps_ref[...]
    eb = (BIAS - (1.0 - BIAS)) * e + (1.0 - BIAS)
    g_ref[...] = jnp.log(eb) - jnp.log(1.0 - eb) + be_ref[0, 0]


_tc_prep = pl.pallas_call(
    _tc_prep_body,
    out_shape=(
        jax.ShapeDtypeStruct((NNODES, 1), _f32),
        jax.ShapeDtypeStruct((RPAD, LANE), _f32),
    ),
)


def _tc_norm_body(dlp0_ref, dlp1_ref, dhp0_ref, dhp1_ref,
                  ilp_ref, ihp_ref, tlp_ref):
    dl = dlp0_ref[...] + dlp1_ref[...] + (1.0 + EOS)
    dh = dhp0_ref[...] + dhp1_ref[...] + (1.0 + EOS)
    ilp_ref[...] = lax.rsqrt(dl)
    ihp_ref[...] = lax.rsqrt(dh)
    tlp_ref[...] = (1.0 + EOS) / dl


_tc_norm = pl.pallas_call(
    _tc_norm_body,
    out_shape=(
        jax.ShapeDtypeStruct((1, NNODES), _f32),
        jax.ShapeDtypeStruct((1, NNODES), _f32),
        jax.ShapeDtypeStruct((1, NNODES), _f32),
    ),
)


_MESH = plsc.VectorSubcoreMesh(core_axis_name="c", subcore_axis_name="s",
                               num_cores=2, num_subcores=16)
_CP = pltpu.CompilerParams(needs_layout_passes=False)


@functools.partial(
    pl.kernel,
    out_type=(
        jax.ShapeDtypeStruct((RPAD * LANE,), _f32),
        jax.ShapeDtypeStruct((RPAD * LANE,), _f32),
        jax.ShapeDtypeStruct((NNODES,), _f32),
        jax.ShapeDtypeStruct((NNODES,), _f32),
        jax.ShapeDtypeStruct((NNODES,), _f32),
        jax.ShapeDtypeStruct((NNODES,), _f32),
        jax.ShapeDtypeStruct((NTILES * NTILES * CAP1 * 16,), _i32),
        jax.ShapeDtypeStruct((NTILES * NTILES * 16,), _i32),
        jax.ShapeDtypeStruct((NTILES * OVC * 16,), _i32),
        jax.ShapeDtypeStruct((NTILES * 16,), _i32),
    ),
    mesh=_MESH,
    compiler_params=_CP,
    scratch_types=[
        pltpu.VMEM((NNODES,), _f32),
        pltpu.VMEM((EPT,), _i32),
        pltpu.VMEM((EPT,), _i32),
        pltpu.VMEM((EPT,), _f32),
        pltpu.VMEM((EPT,), _f32),
        pltpu.VMEM((EPT,), _f32),
        pltpu.VMEM((EPT,), _f32),
        pltpu.VMEM((EPT,), _f32),
        pltpu.VMEM((NTILES * CAP1 * 16,), _i32),
        pltpu.VMEM((NTILES * 16,), _i32),
        pltpu.VMEM((OVC * 16,), _i32),
        pltpu.VMEM((16,), _i32),
        pltpu.VMEM_SHARED((NNODES,), _f32),
        pltpu.VMEM_SHARED((NNODES,), _f32),
        pltpu.SemaphoreType.DMA,
    ],
)
def _sc_weights(q_hbm, srcf_hbm, dstf_hbm, gf_hbm,
                wlp_hbm, whp_hbm, dlp0_hbm, dlp1_hbm, dhp0_hbm, dhp1_hbm,
                bkt_hbm, cnt_hbm, ovl_hbm, ovc_hbm,
                q_v, src_v, dst_v, g_v, wlp_v, whp_v, wlpe_v, whpe_v,
                bbuf_v, bcnt_v, ovbuf_v, ocnt_v,
                sh_lp, sh_hp, sem):
    c = lax.axis_index("c")
    s = lax.axis_index("s")
    wid = s * 2 + c
    ebase = wid * EPT
    nrows = jnp.minimum(TROWS, ROWS - wid * TROWS)
    laneid = lax.iota(_i32, 16)

    cp0 = pltpu.async_copy(q_hbm, q_v, sem)
    cp1 = pltpu.async_copy(srcf_hbm.at[pl.ds(ebase, EPT)], src_v, sem)
    cp2 = pltpu.async_copy(dstf_hbm.at[pl.ds(ebase, EPT)], dst_v, sem)
    cp3 = pltpu.async_copy(gf_hbm.at[pl.ds(ebase, EPT)], g_v, sem)

    def cbody(i, carry):
        bcnt_v[pl.ds(i * 16, 16)] = jnp.zeros((16,), _i32)
        return carry
    lax.fori_loop(0, NTILES, cbody, 0)
    ocnt_v[...] = jnp.zeros((16,), _i32)

    @pl.when(s == 0)
    def _init_shared():
        def zbody(i, carry):
            wlpe_v[pl.ds(i * 16, 16)] = jnp.zeros((16,), _f32)
            return carry
        lax.fori_loop(0, NNODES // 16, zbody, 0)
        pltpu.sync_copy(wlpe_v.at[pl.ds(0, NNODES)], sh_lp)
        pltpu.sync_copy(wlpe_v.at[pl.ds(0, NNODES)], sh_hp)

    cp0.wait()
    cp1.wait()
    cp2.wait()
    cp3.wait()
    plsc.subcore_barrier()

    def row_body(j, carry):
        valid = jnp.broadcast_to(j, (16,)) < nrows
        m = valid.astype(_f32)
        for k in range(LANE // 16):
            fl = pl.ds(j * LANE + k * 16, 16)
            si = src_v[fl]
            di = dst_v[fl]
            qs = plsc.load_gather(q_v, [si])
            qd = plsc.load_gather(q_v, [di])
            x = (g_v[fl] + qs + qd) / TEMP
            w = 1.0 / (1.0 + jnp.exp(-x))
            wlp_v[fl] = w
            whp_v[fl] = 1.0 - w
            wlpe_v[fl] = (w + EOS) * m
            whpe_v[fl] = ((1.0 - w) + EOS) * m
            b = jnp.minimum(si // RT, NTILES - 1)
            fid = si * NNODES + di
            cell = b * 16 + laneid
            bc = plsc.load_gather(bcnt_v, [cell])
            ovf = bc >= CAP1
            okm = valid & jnp.logical_not(ovf)
            plsc.store_scatter(bbuf_v, [b * (CAP1 * 16) + bc * 16 + laneid],
                               fid, mask=okm)
            plsc.store_scatter(bcnt_v, [cell], bc + 1, mask=okm)
            ovm = valid & ovf
            oc = plsc.load_gather(ocnt_v, [laneid])
            plsc.store_scatter(ovbuf_v, [oc * 16 + laneid], fid, mask=ovm)
            plsc.store_scatter(ocnt_v, [laneid],
                               oc + ovm.astype(_i32), mask=ovm)
        return carry

    lax.fori_loop(0, TROWS, row_body, 0)

    pltpu.sync_copy(wlpe_v, sh_lp.at[dst_v], add=True)
    pltpu.sync_copy(whpe_v, sh_hp.at[dst_v], add=True)

    pltpu.sync_copy(wlp_v, wlp_hbm.at[pl.ds(ebase, EPT)])
    pltpu.sync_copy(whp_v, whp_hbm.at[pl.ds(ebase, EPT)])
    pltpu.sync_copy(bbuf_v, bkt_hbm.at[pl.ds(wid * (NTILES * CAP1 * 16),
                                             NTILES * CAP1 * 16)])
    pltpu.sync_copy(bcnt_v, cnt_hbm.at[pl.ds(wid * (NTILES * 16), NTILES * 16)])
    pltpu.sync_copy(ovbuf_v, ovl_hbm.at[pl.ds(wid * (OVC * 16), OVC * 16)])
    pltpu.sync_copy(ocnt_v, ovc_hbm.at[pl.ds(wid * 16, 16)])

    plsc.subcore_barrier()

    @pl.when((s == 0) & (c == 0))
    def _writeback_c0():
        pltpu.sync_copy(sh_lp, dlp0_hbm)
        pltpu.sync_copy(sh_hp, dhp0_hbm)

    @pl.when((s == 0) & (c == 1))
    def _writeback_c1():
        pltpu.sync_copy(sh_lp, dlp1_hbm)
        pltpu.sync_copy(sh_hp, dhp1_hbm)


@functools.partial(
    pl.kernel,
    out_type=(
        jax.ShapeDtypeStruct((RPAD * LANE,), _f32),
        jax.ShapeDtypeStruct((RPAD * LANE,), _f32),
    ),
    mesh=_MESH,
    compiler_params=_CP,
    scratch_types=[
        pltpu.VMEM((NNODES,), _f32),
        pltpu.VMEM((NNODES,), _f32),
        pltpu.VMEM((EPT,), _i32),
        pltpu.VMEM((EPT,), _i32),
        pltpu.VMEM((EPT,), _f32),
        pltpu.VMEM((EPT,), _f32),
        pltpu.VMEM((EPT,), _f32),
        pltpu.VMEM((EPT,), _f32),
        pltpu.SemaphoreType.DMA,
    ],
)
def _sc_norm(ilp_hbm, ihp_hbm, srcf_hbm, dstf_hbm, wlp_hbm, whp_hbm,
             olp_hbm, ohp_hbm,
             ilp_v, ihp_v, src_v, dst_v, wlp_v, whp_v, olp_v, ohp_v, sem):
    c = lax.axis_index("c")
    s = lax.axis_index("s")
    wid = s * 2 + c
    ebase = wid * EPT

    cp0 = pltpu.async_copy(ilp_hbm, ilp_v, sem)
    cp1 = pltpu.async_copy(ihp_hbm, ihp_v, sem)
    cp2 = pltpu.async_copy(srcf_hbm.at[pl.ds(ebase, EPT)], src_v, sem)
    cp3 = pltpu.async_copy(dstf_hbm.at[pl.ds(ebase, EPT)], dst_v, sem)
    cp4 = pltpu.async_copy(wlp_hbm.at[pl.ds(ebase, EPT)], wlp_v, sem)
    cp5 = pltpu.async_copy(whp_hbm.at[pl.ds(ebase, EPT)], whp_v, sem)
    cp0.wait()
    cp1.wait()
    cp2.wait()
    cp3.wait()
    cp4.wait()
    cp5.wait()

    def grp_body(i, carry):
        fl = pl.ds(i * 16, 16)
        si = src_v[fl]
        di = dst_v[fl]
        ils = plsc.load_gather(ilp_v, [si])
        ild = plsc.load_gather(ilp_v, [di])
        ihs = plsc.load_gather(ihp_v, [si])
        ihd = plsc.load_gather(ihp_v, [di])
        olp_v[fl] = (wlp_v[fl] + EOS) * ils * ild
        ohp_v[fl] = (-ALPHA) * ((whp_v[fl] + EOS) * ihs * ihd)
        return carry

    lax.fori_loop(0, EPT // 16, grp_body, 0)

    pltpu.sync_copy(olp_v, olp_hbm.at[pl.ds(ebase, EPT)])
    pltpu.sync_copy(ohp_v, ohp_hbm.at[pl.ds(ebase, EPT)])


@functools.partial(
    pl.kernel,
    out_type=jax.ShapeDtypeStruct((NNODES, NNODES), _f32),
    mesh=_MESH,
    compiler_params=_CP,
    scratch_types=[
        pltpu.VMEM((8, NNODES), _f32),
        pltpu.VMEM((NU_MAX * CAP2 * 16,), _i32),
        pltpu.VMEM((NU_MAX * 16,), _i32),
        pltpu.VMEM((OVC * 16,), _i32),
        pltpu.VMEM((16,), _i32),
        pltpu.VMEM((CAP1 * 16,), _i32),
        pltpu.VMEM((16,), _i32),
        pltpu.VMEM((16,), _i32),
        pltpu.SemaphoreType.DMA,
    ],
)
def _sc_adj(bkt_hbm, cnt_hbm, ovl_hbm, ovc_hbm, adj_hbm,
            row_v, ubuf_v, ucnt_v, lov_v, locnt_v,
            cell_v, c16_v, ovch_v, sem):
    c = lax.axis_index("c")
    s = lax.axis_index("s")
    w = s * 2 + c
    r0 = w * RT
    nr = jnp.where(w == NTILES - 1, NNODES - RT * (NTILES - 1), RT)
    nu = nr // 8
    laneid = lax.iota(_i32, 16)

    def zc(i, carry):
        ucnt_v[pl.ds(i * 16, 16)] = jnp.zeros((16,), _i32)
        return carry
    lax.fori_loop(0, NU_MAX, zc, 0)
    locnt_v[...] = jnp.zeros((16,), _i32)

    def insert(fid, valid):
        src = fid // NNODES
        u = jnp.clip((src - r0) // 8, 0, NU_MAX - 1)
        cellix = u * 16 + laneid
        uc = plsc.load_gather(ucnt_v, [cellix])
        ovf = uc >= CAP2
        okm = valid & jnp.logical_not(ovf)
        plsc.store_scatter(ubuf_v, [u * (CAP2 * 16) + uc * 16 + laneid],
                           fid, mask=okm)
        plsc.store_scatter(ucnt_v, [cellix], uc + 1, mask=okm)
        ovm = valid & ovf
        oc = plsc.load_gather(locnt_v, [laneid])
        plsc.store_scatter(lov_v, [oc * 16 + laneid], fid, mask=ovm)
        plsc.store_scatter(locnt_v, [laneid], oc + ovm.astype(_i32), mask=ovm)

    def prod_body(t, carry):
        pltpu.sync_copy(cnt_hbm.at[pl.ds(t * (NTILES * 16) + w * 16, 16)],
                        c16_v)
        pltpu.sync_copy(
            bkt_hbm.at[pl.ds(t * (NTILES * CAP1 * 16) + w * (CAP1 * 16),
                             CAP1 * 16)], cell_v)
        cnt16 = c16_v[...]
        maxc = lax.reduce_max(cnt16, (0,))

        def slot_body(q, carry2):
            fid = cell_v[pl.ds(q * 16, 16)]
            insert(fid, jnp.broadcast_to(q, (16,)) < cnt16)
            return carry2
        lax.fori_loop(0, maxc, slot_body, 0)
        return carry

    lax.fori_loop(0, NTILES, prod_body, 0)

    def pov_body(t, carry):
        pltpu.sync_copy(ovc_hbm.at[pl.ds(t * 16, 16)], c16_v)
        ocnt16 = c16_v[...]
        maxo = lax.reduce_max(ocnt16, (0,))

        def oslot_body(q, carry2):
            pltpu.sync_copy(ovl_hbm.at[pl.ds(t * (OVC * 16) + q * 16, 16)],
                            ovch_v)
            fid = ovch_v[...]
            src = fid // NNODES
            mine = (jnp.broadcast_to(q, (16,)) < ocnt16) \
                & (src >= r0) & (src < r0 + nr)
            insert(fid, mine)
            return carry2
        lax.fori_loop(0, maxo, oslot_body, 0)
        return carry

    lax.fori_loop(0, NTILES, pov_body, 0)

    for r in range(8):
        def zb(i, carry, r=r):
            row_v[r, pl.ds(i * 16, 16)] = jnp.zeros((16,), _f32)
            return carry
        lax.fori_loop(0, NNODES // 16, zb, 0)

    ones16 = jnp.full((16,), 1.0, _f32)
    zeros16 = jnp.zeros((16,), _f32)

    def unit_body(u, carry):
        cnt16 = ucnt_v[pl.ds(u * 16, 16)]
        maxc = lax.reduce_max(cnt16, (0,))
        ubase = r0 + u * 8

        def scat(q, carry2, val):
            fid = ubuf_v[pl.ds(u * (CAP2 * 16) + q * 16, 16)]
            src = fid // NNODES
            dstc = fid - src * NNODES
            valid = jnp.broadcast_to(q, (16,)) < cnt16
            plsc.store_scatter(row_v, [jnp.clip(src - ubase, 0, 7), dstc],
                               val, mask=valid)
            return carry2

        def ovscat(q, carry2, val):
            fid = lov_v[pl.ds(q * 16, 16)]
            src = fid // NNODES
            dstc = fid - src * NNODES
            mine = (jnp.broadcast_to(q, (16,)) < locnt_v[...]) \
                & (src >= ubase) & (src < ubase + 8)
            plsc.store_scatter(row_v, [jnp.clip(src - ubase, 0, 7), dstc],
                               val, mask=mine)
            return carry2

        maxo = lax.reduce_max(locnt_v[...], (0,))
        lax.fori_loop(0, maxc, functools.partial(scat, val=ones16), 0)
        lax.fori_loop(0, maxo, functools.partial(ovscat, val=ones16), 0)

        pltpu.sync_copy(row_v, adj_hbm.at[pl.ds(ubase, 8)])

        lax.fori_loop(0, maxc, functools.partial(scat, val=zeros16), 0)
        lax.fori_loop(0, maxo, functools.partial(ovscat, val=zeros16), 0)
        return carry

    lax.fori_loop(0, nu, unit_body, 0)


def kernel(features, edges, eps, W1, b1, W_edge, b_edge):
    src = edges[0].astype(_i32)
    dst = edges[1].astype(_i32)
    pad = ((0, RPAD - ROWS), (0, 0))
    srcf = jnp.pad(src.reshape(ROWS, LANE), pad).reshape(RPAD * LANE)
    dstf = jnp.pad(dst.reshape(ROWS, LANE), pad).reshape(RPAD * LANE)
    eps2 = jnp.pad(eps.reshape(ROWS, LANE), pad)

    q2, g2 = _tc_prep(features, W1, b1.reshape(1, HID), W_edge,
                      b_edge.reshape(1, 1), eps2)
    q = q2.reshape(NNODES)
    gf = g2.reshape(RPAD * LANE)

    (wlpf, whpf, dlp0, dlp1, dhp0, dhp1,
     bkt, cnt, ovl, ovc) = _sc_weights(q, srcf, dstf, gf)
    ilp, ihp, tlp = _tc_norm(dlp0.reshape(1, NNODES), dlp1.reshape(1, NNODES),
                             dhp0.reshape(1, NNODES), dhp1.reshape(1, NNODES))

    olpf, ohpf = _sc_norm(ilp.reshape(NNODES), ihp.reshape(NNODES),
                          srcf, dstf, wlpf, whpf)
    adj = _sc_adj(bkt, cnt, ovl, ovc)

    weights_lp = wlpf[:NEDGES]
    weights_hp = whpf[:NEDGES]
    w_lp_norm = jnp.concatenate([olpf[:NEDGES], tlp.reshape(NNODES)])
    w_hp_norm = jnp.concatenate([ohpf[:NEDGES], jnp.ones((NNODES,), _f32)])
    return (w_lp_norm, w_hp_norm, weights_lp, weights_hp, adj)


# speedup vs baseline: 2.7176x; 1.5123x over previous
import functools

import jax
import jax.numpy as jnp
from jax import lax
from jax.experimental import pallas as pl
from jax.experimental.pallas import tpu as pltpu
from jax.experimental.pallas import tpu_sc as plsc

EOS = 1e-10
NNODES = 10000
NEDGES = 320000
IN_DIM = 128
HID = 128
ALPHA = 1.0
TEMP = 1.0
BIAS = 0.0001

LANE = 128
ROWS = NEDGES // LANE
NTILES = 32
TROWS = 80
RPAD = NTILES * TROWS
EPT = TROWS * LANE

RT = 312
NU_MAX = 41
CAP1 = 64
CAP2 = 32
OVC = EPT // 16
CELLW = 16 + CAP1 * 16
BKTW = NTILES * CELLW
QP = 8

_f32 = jnp.float32
_i32 = jnp.int32


def _tc_prep_body(f_ref, w1_ref, b1_ref, we_ref, be_ref, eps_ref, q_ref, g_ref):
    h = jnp.dot(f_ref[...], w1_ref[...], preferred_element_type=_f32)
    h = jnp.maximum(h + b1_ref[...], 0.0)
    w2 = 0.5 * (we_ref[:HID, :] + we_ref[HID:, :])
    q_ref[...] = jnp.dot(h, w2, preferred_element_type=_f32)
    e = eps_ref[...]
    eb = (BIAS - (1.0 - BIAS)) * e + (1.0 - BIAS)
    g_ref[...] = jnp.log(eb) - jnp.log(1.0 - eb) + be_ref[0, 0]


_tc_prep = pl.pallas_call(
    _tc_prep_body,
    out_shape=(
        jax.ShapeDtypeStruct((NNODES, 1), _f32),
        jax.ShapeDtypeStruct((RPAD, LANE), _f32),
    ),
)


def _tc_norm_body(dlp0_ref, dlp1_ref, dhp0_ref, dhp1_ref,
                  ilp_ref, ihp_ref, tlp_ref):
    dl = dlp0_ref[...] + dlp1_ref[...] + (1.0 + EOS)
    dh = dhp0_ref[...] + dhp1_ref[...] + (1.0 + EOS)
    ilp_ref[...] = lax.rsqrt(dl)
    ihp_ref[...] = lax.rsqrt(dh)
    tlp_ref[...] = (1.0 + EOS) / dl


_tc_norm = pl.pallas_call(
    _tc_norm_body,
    out_shape=(
        jax.ShapeDtypeStruct((1, NNODES), _f32),
        jax.ShapeDtypeStruct((1, NNODES), _f32),
        jax.ShapeDtypeStruct((1, NNODES), _f32),
    ),
)


_MESH = plsc.VectorSubcoreMesh(core_axis_name="c", subcore_axis_name="s",
                               num_cores=2, num_subcores=16)
_CP = pltpu.CompilerParams(needs_layout_passes=False)


@functools.partial(
    pl.kernel,
    out_type=(
        jax.ShapeDtypeStruct((RPAD * LANE,), _f32),
        jax.ShapeDtypeStruct((RPAD * LANE,), _f32),
        jax.ShapeDtypeStruct((NNODES,), _f32),
        jax.ShapeDtypeStruct((NNODES,), _f32),
        jax.ShapeDtypeStruct((NNODES,), _f32),
        jax.ShapeDtypeStruct((NNODES,), _f32),
        jax.ShapeDtypeStruct((NTILES * BKTW,), _i32),
        jax.ShapeDtypeStruct((NTILES * OVC * 16,), _i32),
        jax.ShapeDtypeStruct((NTILES * 16,), _i32),
    ),
    mesh=_MESH,
    compiler_params=_CP,
    scratch_types=[
        pltpu.VMEM((NNODES,), _f32),
        pltpu.VMEM((EPT,), _i32),
        pltpu.VMEM((EPT,), _i32),
        pltpu.VMEM((EPT,), _f32),
        pltpu.VMEM((EPT,), _f32),
        pltpu.VMEM((EPT,), _f32),
        pltpu.VMEM((EPT,), _f32),
        pltpu.VMEM((EPT,), _f32),
        pltpu.VMEM((NTILES * CELLW,), _i32),
        pltpu.VMEM((NTILES * 16,), _i32),
        pltpu.VMEM((OVC * 16,), _i32),
        pltpu.VMEM((16,), _i32),
        pltpu.VMEM_SHARED((NNODES,), _f32),
        pltpu.VMEM_SHARED((NNODES,), _f32),
        pltpu.SemaphoreType.DMA,
    ],
)
def _sc_weights(q_hbm, srcf_hbm, dstf_hbm, gf_hbm,
                wlp_hbm, whp_hbm, dlp0_hbm, dlp1_hbm, dhp0_hbm, dhp1_hbm,
                bkt_hbm, ovl_hbm, ovc_hbm,
                q_v, src_v, dst_v, g_v, wlp_v, whp_v, wlpe_v, whpe_v,
                bbuf_v, bcnt_v, ovbuf_v, ocnt_v,
                sh_lp, sh_hp, sem):
    c = lax.axis_index("c")
    s = lax.axis_index("s")
    wid = s * 2 + c
    ebase = wid * EPT
    nrows = jnp.minimum(TROWS, ROWS - wid * TROWS)
    laneid = lax.iota(_i32, 16)

    cp0 = pltpu.async_copy(q_hbm, q_v, sem)
    cp1 = pltpu.async_copy(srcf_hbm.at[pl.ds(ebase, EPT)], src_v, sem)
    cp2 = pltpu.async_copy(dstf_hbm.at[pl.ds(ebase, EPT)], dst_v, sem)
    cp3 = pltpu.async_copy(gf_hbm.at[pl.ds(ebase, EPT)], g_v, sem)

    def cbody(i, carry):
        bcnt_v[pl.ds(i * 16, 16)] = jnp.zeros((16,), _i32)
        return carry
    lax.fori_loop(0, NTILES, cbody, 0)
    ocnt_v[...] = jnp.zeros((16,), _i32)

    @pl.when(s == 0)
    def _init_shared():
        def zbody(i, carry):
            wlpe_v[pl.ds(i * 16, 16)] = jnp.zeros((16,), _f32)
            return carry
        lax.fori_loop(0, NNODES // 16, zbody, 0)
        pltpu.sync_copy(wlpe_v.at[pl.ds(0, NNODES)], sh_lp)
        pltpu.sync_copy(wlpe_v.at[pl.ds(0, NNODES)], sh_hp)

    cp0.wait()
    cp1.wait()
    cp2.wait()
    cp3.wait()
    plsc.subcore_barrier()

    def row_body(j, carry):
        valid = jnp.broadcast_to(j, (16,)) < nrows
        m = valid.astype(_f32)
        for k in range(LANE // 16):
            fl = pl.ds(j * LANE + k * 16, 16)
            si = src_v[fl]
            di = dst_v[fl]
            qs = plsc.load_gather(q_v, [si])
            qd = plsc.load_gather(q_v, [di])
            x = (g_v[fl] + qs + qd) / TEMP
            w = 1.0 / (1.0 + jnp.exp(-x))
            wlp_v[fl] = w
            whp_v[fl] = 1.0 - w
            wlpe_v[fl] = (w + EOS) * m
            whpe_v[fl] = ((1.0 - w) + EOS) * m
            b = jnp.minimum(si // RT, NTILES - 1)
            lfid = (si - b * RT) * 16384 + di
            cell = b * 16 + laneid
            bc = plsc.load_gather(bcnt_v, [cell])
            ovf = bc >= CAP1
            okm = valid & jnp.logical_not(ovf)
            plsc.store_scatter(bbuf_v,
                               [b * CELLW + 16 + bc * 16 + laneid],
                               lfid, mask=okm)
            plsc.store_scatter(bcnt_v, [cell], bc + 1, mask=okm)
            ovm = valid & ovf
            fid = si * NNODES + di
            oc = plsc.load_gather(ocnt_v, [laneid])
            plsc.store_scatter(ovbuf_v, [oc * 16 + laneid], fid, mask=ovm)
            plsc.store_scatter(ocnt_v, [laneid],
                               oc + ovm.astype(_i32), mask=ovm)
        return carry

    lax.fori_loop(0, TROWS, row_body, 0)

    for bh in range(NTILES):
        bbuf_v[pl.ds(bh * CELLW, 16)] = bcnt_v[pl.ds(bh * 16, 16)]

    pltpu.sync_copy(wlpe_v, sh_lp.at[dst_v], add=True)
    pltpu.sync_copy(whpe_v, sh_hp.at[dst_v], add=True)
    cps = [
        pltpu.async_copy(wlp_v, wlp_hbm.at[pl.ds(ebase, EPT)], sem),
        pltpu.async_copy(whp_v, whp_hbm.at[pl.ds(ebase, EPT)], sem),
        pltpu.async_copy(ovbuf_v, ovl_hbm.at[pl.ds(wid * (OVC * 16), OVC * 16)],
                         sem),
        pltpu.async_copy(ocnt_v, ovc_hbm.at[pl.ds(wid * 16, 16)], sem),
    ]
    for bh in range(NTILES):
        cps.append(pltpu.async_copy(
            bbuf_v.at[pl.ds(bh * CELLW, CELLW)],
            bkt_hbm.at[pl.ds(bh * BKTW + wid * CELLW, CELLW)], sem))
        if len(cps) >= 8:
            for cp in cps:
                cp.wait()
            cps = []
    for cp in cps:
        cp.wait()

    plsc.subcore_barrier()

    @pl.when((s == 0) & (c == 0))
    def _writeback_c0():
        pltpu.sync_copy(sh_lp, dlp0_hbm)
        pltpu.sync_copy(sh_hp, dhp0_hbm)

    @pl.when((s == 0) & (c == 1))
    def _writeback_c1():
        pltpu.sync_copy(sh_lp, dlp1_hbm)
        pltpu.sync_copy(sh_hp, dhp1_hbm)


@functools.partial(
    pl.kernel,
    out_type=(
        jax.ShapeDtypeStruct((RPAD * LANE,), _f32),
        jax.ShapeDtypeStruct((RPAD * LANE,), _f32),
    ),
    mesh=_MESH,
    compiler_params=_CP,
    scratch_types=[
        pltpu.VMEM((NNODES,), _f32),
        pltpu.VMEM((NNODES,), _f32),
        pltpu.VMEM((EPT,), _i32),
        pltpu.VMEM((EPT,), _i32),
        pltpu.VMEM((EPT,), _f32),
        pltpu.VMEM((EPT,), _f32),
        pltpu.VMEM((EPT,), _f32),
        pltpu.VMEM((EPT,), _f32),
        pltpu.SemaphoreType.DMA,
    ],
)
def _sc_norm(ilp_hbm, ihp_hbm, srcf_hbm, dstf_hbm, wlp_hbm, whp_hbm,
             olp_hbm, ohp_hbm,
             ilp_v, ihp_v, src_v, dst_v, wlp_v, whp_v, olp_v, ohp_v, sem):
    c = lax.axis_index("c")
    s = lax.axis_index("s")
    wid = s * 2 + c
    ebase = wid * EPT

    cps = [
        pltpu.async_copy(ilp_hbm, ilp_v, sem),
        pltpu.async_copy(ihp_hbm, ihp_v, sem),
        pltpu.async_copy(srcf_hbm.at[pl.ds(ebase, EPT)], src_v, sem),
        pltpu.async_copy(dstf_hbm.at[pl.ds(ebase, EPT)], dst_v, sem),
        pltpu.async_copy(wlp_hbm.at[pl.ds(ebase, EPT)], wlp_v, sem),
        pltpu.async_copy(whp_hbm.at[pl.ds(ebase, EPT)], whp_v, sem),
    ]
    for cp in cps:
        cp.wait()

    def grp_body(i, carry):
        fl = pl.ds(i * 16, 16)
        si = src_v[fl]
        di = dst_v[fl]
        ils = plsc.load_gather(ilp_v, [si])
        ild = plsc.load_gather(ilp_v, [di])
        ihs = plsc.load_gather(ihp_v, [si])
        ihd = plsc.load_gather(ihp_v, [di])
        olp_v[fl] = (wlp_v[fl] + EOS) * ils * ild
        ohp_v[fl] = (-ALPHA) * ((whp_v[fl] + EOS) * ihs * ihd)
        return carry

    lax.fori_loop(0, EPT // 16, grp_body, 0)

    pltpu.sync_copy(olp_v, olp_hbm.at[pl.ds(ebase, EPT)])
    pltpu.sync_copy(ohp_v, ohp_hbm.at[pl.ds(ebase, EPT)])


@functools.partial(
    pl.kernel,
    out_type=jax.ShapeDtypeStruct((NNODES, NNODES), _f32),
    mesh=_MESH,
    compiler_params=_CP,
    scratch_types=[
        pltpu.VMEM((8, NNODES), _f32),
        pltpu.VMEM((NU_MAX * CAP2 * 16,), _i32),
        pltpu.VMEM((NU_MAX * 16,), _i32),
        pltpu.VMEM((OVC * 16,), _i32),
        pltpu.VMEM((16,), _i32),
        pltpu.VMEM((QP * CELLW,), _i32),
        pltpu.VMEM((QP * CELLW,), _i32),
        pltpu.VMEM((NTILES * 16,), _i32),
        pltpu.VMEM((16,), _i32),
        pltpu.SemaphoreType.DMA,
        pltpu.SemaphoreType.DMA,
    ],
)
def _sc_adj(bkt_hbm, ovl_hbm, ovc_hbm, adj_hbm,
            row_v, ubuf_v, ucnt_v, lov_v, locnt_v,
            stga_v, stgb_v, ovcall_v, ovch_v, sem, sem2):
    c = lax.axis_index("c")
    s = lax.axis_index("s")
    w = s * 2 + c
    r0 = w * RT
    nr = jnp.where(w == NTILES - 1, NNODES - RT * (NTILES - 1), RT)
    nu = nr // 8
    laneid = lax.iota(_i32, 16)

    def zc(i, carry):
        ucnt_v[pl.ds(i * 16, 16)] = jnp.zeros((16,), _i32)
        return carry
    lax.fori_loop(0, NU_MAX, zc, 0)
    locnt_v[...] = jnp.zeros((16,), _i32)

    cpo = pltpu.async_copy(ovc_hbm, ovcall_v, sem2)

    def insert(lfid, valid):
        u = jnp.clip(lax.shift_right_logical(lfid, 17), 0, NU_MAX - 1)
        cellix = u * 16 + laneid
        uc = plsc.load_gather(ucnt_v, [cellix])
        ovf = uc >= CAP2
        okm = valid & jnp.logical_not(ovf)
        plsc.store_scatter(ubuf_v, [u * (CAP2 * 16) + uc * 16 + laneid],
                           lfid, mask=okm)
        plsc.store_scatter(ucnt_v, [cellix], uc + 1, mask=okm)
        ovm = valid & ovf
        oc = plsc.load_gather(locnt_v, [laneid])
        plsc.store_scatter(lov_v, [oc * 16 + laneid], lfid,
                           mask=ovm & (oc < OVC))
        plsc.store_scatter(locnt_v, [laneid], oc + 1, mask=ovm)

    nq = NTILES // QP
    descs = [pltpu.async_copy(
        bkt_hbm.at[pl.ds(w * BKTW, QP * CELLW)], stga_v, sem)]
    for qi in range(nq):
        cur = stga_v if qi % 2 == 0 else stgb_v
        descs.pop(0).wait()
        if qi + 1 < nq:
            nxt = stgb_v if qi % 2 == 0 else stga_v
            descs.append(pltpu.async_copy(
                bkt_hbm.at[pl.ds(w * BKTW + (qi + 1) * (QP * CELLW),
                                 QP * CELLW)], nxt, sem))
        for tq in range(QP):
            cnt16 = cur[pl.ds(tq * CELLW, 16)]
            maxc = jnp.minimum(lax.reduce_max(cnt16, (0,)), CAP1)

            def slot_body(q, carry2, cur=cur, tq=tq, cnt16=cnt16):
                lfid = cur[pl.ds(tq * CELLW + 16 + q * 16, 16)]
                insert(lfid, jnp.broadcast_to(q, (16,)) < cnt16)
                return carry2
            lax.fori_loop(0, maxc, slot_body, 0)

    cpo.wait()

    def pov_body(t, carry):
        ocnt16 = ovcall_v[pl.ds(t * 16, 16)]
        maxo = jnp.minimum(lax.reduce_max(ocnt16, (0,)), OVC)

        def oslot_body(q, carry2):
            pltpu.sync_copy(ovl_hbm.at[pl.ds(t * (OVC * 16) + q * 16, 16)],
                            ovch_v)
            fid = ovch_v[...]
            src = fid // NNODES
            mine = (jnp.broadcast_to(q, (16,)) < ocnt16) \
                & (src >= r0) & (src < r0 + nr)
            insert((src - r0) * 16384 + (fid - src * NNODES), mine)
            return carry2
        lax.fori_loop(0, maxo, oslot_body, 0)
        return carry

    lax.fori_loop(0, NTILES, pov_body, 0)

    spilled = lax.reduce_max(locnt_v[...], (0,)) > OVC

    for r in range(8):
        def zb(i, carry, r=r):
            row_v[r, pl.ds(i * 16, 16)] = jnp.zeros((16,), _f32)
            return carry
        lax.fori_loop(0, NNODES // 16, zb, 0)

    ones16 = jnp.full((16,), 1.0, _f32)
    zeros16 = jnp.zeros((16,), _f32)

    def rescan(u, val):
        def rq_body(t, carry):
            pltpu.sync_copy(bkt_hbm.at[pl.ds(w * BKTW + t * CELLW, CELLW)],
                            stga_v.at[pl.ds(0, CELLW)])
            cnt16 = stga_v[pl.ds(0, 16)]
            maxc = jnp.minimum(lax.reduce_max(cnt16, (0,)), CAP1)

            def rslot(q, carry2):
                lfid = stga_v[pl.ds(16 + q * 16, 16)]
                ok = (jnp.broadcast_to(q, (16,)) < cnt16) \
                    & (lax.shift_right_logical(lfid, 17) == u)
                plsc.store_scatter(
                    row_v,
                    [lax.shift_right_logical(lfid, 14) & 7, lfid & 16383],
                    val, mask=ok)
                return carry2
            lax.fori_loop(0, maxc, rslot, 0)

            ocnt16 = ovcall_v[pl.ds(t * 16, 16)]
            maxo = jnp.minimum(lax.reduce_max(ocnt16, (0,)), OVC)

            def roslot(q, carry2):
                pltpu.sync_copy(
                    ovl_hbm.at[pl.ds(t * (OVC * 16) + q * 16, 16)], ovch_v)
                fid = ovch_v[...]
                src = fid // NNODES
                ok = (jnp.broadcast_to(q, (16,)) < ocnt16) \
                    & (src >= r0 + u * 8) & (src < r0 + u * 8 + 8)
                plsc.store_scatter(
                    row_v, [jnp.clip(src - (r0 + u * 8), 0, 7),
                            fid - src * NNODES], val, mask=ok)
                return carry2
            lax.fori_loop(0, maxo, roslot, 0)
            return carry
        lax.fori_loop(0, NTILES, rq_body, 0)

    def unit_body(u, carry):
        cnt16 = ucnt_v[pl.ds(u * 16, 16)]
        maxc = jnp.minimum(lax.reduce_max(cnt16, (0,)), CAP2)
        locnt16 = jnp.minimum(locnt_v[...], OVC)
        maxo = lax.reduce_max(locnt16, (0,))

        def scat(q, carry2, val):
            lfid = ubuf_v[pl.ds(u * (CAP2 * 16) + q * 16, 16)]
            valid = jnp.broadcast_to(q, (16,)) < cnt16
            plsc.store_scatter(
                row_v, [lax.shift_right_logical(lfid, 14) & 7, lfid & 16383],
                val, mask=valid)
            return carry2

        def ovscat(q, carry2, val):
            lfid = lov_v[pl.ds(q * 16, 16)]
            ok = (jnp.broadcast_to(q, (16,)) < locnt16) \
                & (lax.shift_right_logical(lfid, 17) == u)
            plsc.store_scatter(
                row_v, [lax.shift_right_logical(lfid, 14) & 7, lfid & 16383],
                val, mask=ok)
            return carry2

        lax.fori_loop(0, maxc, functools.partial(scat, val=ones16), 0)
        lax.fori_loop(0, maxo, functools.partial(ovscat, val=ones16), 0)

        @pl.when(spilled)
        def _slow_on():
            rescan(u, ones16)

        pltpu.sync_copy(row_v, adj_hbm.at[pl.ds(r0 + u * 8, 8)])

        lax.fori_loop(0, maxc, functools.partial(scat, val=zeros16), 0)
        lax.fori_loop(0, maxo, functools.partial(ovscat, val=zeros16), 0)

        @pl.when(spilled)
        def _slow_off():
            rescan(u, zeros16)
        return carry

    lax.fori_loop(0, nu, unit_body, 0)


def kernel(features, edges, eps, W1, b1, W_edge, b_edge):
    src = edges[0].astype(_i32)
    dst = edges[1].astype(_i32)
    pad = ((0, RPAD - ROWS), (0, 0))
    srcf = jnp.pad(src.reshape(ROWS, LANE), pad).reshape(RPAD * LANE)
    dstf = jnp.pad(dst.reshape(ROWS, LANE), pad).reshape(RPAD * LANE)
    eps2 = jnp.pad(eps.reshape(ROWS, LANE), pad)

    q2, g2 = _tc_prep(features, W1, b1.reshape(1, HID), W_edge,
                      b_edge.reshape(1, 1), eps2)
    q = q2.reshape(NNODES)
    gf = g2.reshape(RPAD * LANE)

    (wlpf, whpf, dlp0, dlp1, dhp0, dhp1,
     bkt, ovl, ovc) = _sc_weights(q, srcf, dstf, gf)
    ilp, ihp, tlp = _tc_norm(dlp0.reshape(1, NNODES), dlp1.reshape(1, NNODES),
                             dhp0.reshape(1, NNODES), dhp1.reshape(1, NNODES))

    olpf, ohpf = _sc_norm(ilp.reshape(NNODES), ihp.reshape(NNODES),
                          srcf, dstf, wlpf, whpf)
    adj = _sc_adj(bkt, ovl, ovc)

    weights_lp = wlpf[:NEDGES]
    weights_hp = whpf[:NEDGES]
    w_lp_norm = jnp.concatenate([olpf[:NEDGES], tlp.reshape(NNODES)])
    w_hp_norm = jnp.concatenate([ohpf[:NEDGES], jnp.ones((NNODES,), _f32)])
    return (w_lp_norm, w_hp_norm, weights_lp, weights_hp, adj)
